# baseline (speedup 1.0000x reference)
import functools

import jax
import jax.numpy as jnp
from jax import lax
from jax.experimental import pallas as pl
from jax.experimental.pallas import tpu as pltpu
from jax.experimental.pallas import tpu_sc as plsc

S = 64.0
M = 0.4

_NC = 2
_NS = 16
_NW = _NC * _NS
_L = 16


def _sc_margin_body(n_rows, n_cols, cos_flat_hbm, label_hbm, adj_hbm,
                    lbl_v, idx_v, vals_v, sem):
    rows_per_w = n_rows // _NW
    wid = lax.axis_index("s") * _NC + lax.axis_index("c")
    base = wid * rows_per_w
    pltpu.sync_copy(label_hbm.at[pl.ds(base, rows_per_w)], lbl_v)
    for k in range(rows_per_w // _L):
        lbl = lbl_v[pl.ds(k * _L, _L)]
        safe = jnp.where(lbl >= 0, lbl, 0)
        rows = lax.iota(jnp.int32, _L) + (base + k * _L)
        idx_v[pl.ds(k * _L, _L)] = rows * n_cols + safe
    pltpu.async_copy(cos_flat_hbm.at[idx_v], vals_v, sem).wait()
    for k in range(rows_per_w // _L):
        v = vals_v[pl.ds(k * _L, _L)]
        lbl = lbl_v[pl.ds(k * _L, _L)]
        margin = jnp.where(lbl >= 0, jnp.float32(M), jnp.float32(0.0))
        vals_v[pl.ds(k * _L, _L)] = (v - margin) * jnp.float32(S)
    pltpu.sync_copy(vals_v, adj_hbm.at[pl.ds(base, rows_per_w)])


def _tc_inject_body(n_cols_blk, cos_ref, lbl_ref, adj_ref, out_ref):
    col0 = pl.program_id(1) * n_cols_blk
    shape = cos_ref.shape
    cols = lax.broadcasted_iota(jnp.int32, shape, 1) + col0
    out_ref[...] = jnp.where(cols == lbl_ref[...], adj_ref[...],
                             cos_ref[...] * jnp.float32(S))


def kernel(cosine, label):
    n_rows, n_cols = cosine.shape
    rows_per_w = n_rows // _NW

    mesh = plsc.VectorSubcoreMesh(core_axis_name="c", subcore_axis_name="s",
                                  num_cores=_NC, num_subcores=_NS)
    sc_fn = pl.kernel(
        functools.partial(_sc_margin_body, n_rows, n_cols),
        out_type=jax.ShapeDtypeStruct((n_rows,), jnp.float32),
        mesh=mesh,
        scratch_types=[
            pltpu.VMEM((rows_per_w,), jnp.int32),
            pltpu.VMEM((rows_per_w,), jnp.int32),
            pltpu.VMEM((rows_per_w,), jnp.float32),
            pltpu.SemaphoreType.DMA,
        ],
    )
    adj = sc_fn(cosine.reshape(-1), label)

    blk_r, blk_c = 256, 2048
    grid = (n_rows // blk_r, pl.cdiv(n_cols, blk_c))
    out = pl.pallas_call(
        functools.partial(_tc_inject_body, blk_c),
        grid=grid,
        in_specs=[
            pl.BlockSpec((blk_r, blk_c), lambda i, j: (i, j)),
            pl.BlockSpec((blk_r, 1), lambda i, j: (i, 0)),
            pl.BlockSpec((blk_r, 1), lambda i, j: (i, 0)),
        ],
        out_specs=pl.BlockSpec((blk_r, blk_c), lambda i, j: (i, j)),
        out_shape=jax.ShapeDtypeStruct((n_rows, n_cols), jnp.float32),
        compiler_params=pltpu.CompilerParams(
            dimension_semantics=("parallel", "parallel"),
        ),
    )(cosine, label.reshape(n_rows, 1), adj.reshape(n_rows, 1))
    return out


# speedup vs baseline: 1.0111x; 1.0111x over previous
import functools

import jax
import jax.numpy as jnp
from jax import lax
from jax.experimental import pallas as pl
from jax.experimental.pallas import tpu as pltpu
from jax.experimental.pallas import tpu_sc as plsc

S = 64.0
M = 0.4

_NC = 2
_NS = 16
_NW = _NC * _NS
_L = 16


def _sc_margin_body(n_rows, n_cols, cos_flat_hbm, label_hbm, adj_hbm,
                    lbl_v, idx_v, vals_v, sem):
    rows_per_w = n_rows // _NW
    wid = lax.axis_index("s") * _NC + lax.axis_index("c")
    base = wid * rows_per_w
    pltpu.sync_copy(label_hbm.at[pl.ds(base, rows_per_w)], lbl_v)
    for k in range(rows_per_w // _L):
        lbl = lbl_v[pl.ds(k * _L, _L)]
        safe = jnp.where(lbl >= 0, lbl, 0)
        rows = lax.iota(jnp.int32, _L) + (base + k * _L)
        idx_v[pl.ds(k * _L, _L)] = rows * n_cols + safe
    pltpu.async_copy(cos_flat_hbm.at[idx_v], vals_v, sem).wait()
    for k in range(rows_per_w // _L):
        v = vals_v[pl.ds(k * _L, _L)]
        lbl = lbl_v[pl.ds(k * _L, _L)]
        margin = jnp.where(lbl >= 0, jnp.float32(M), jnp.float32(0.0))
        vals_v[pl.ds(k * _L, _L)] = (v - margin) * jnp.float32(S)
    pltpu.sync_copy(vals_v, adj_hbm.at[pl.ds(base, rows_per_w)])


def _tc_inject_body(n_cols_blk, cos_ref, lbl_ref, adj_ref, out_ref):
    col0 = pl.program_id(1) * n_cols_blk
    shape = cos_ref.shape
    cols = lax.broadcasted_iota(jnp.int32, shape, 1) + col0
    out_ref[...] = jnp.where(cols == lbl_ref[...], adj_ref[...],
                             cos_ref[...] * jnp.float32(S))


def kernel(cosine, label):
    n_rows, n_cols = cosine.shape
    rows_per_w = n_rows // _NW

    mesh = plsc.VectorSubcoreMesh(core_axis_name="c", subcore_axis_name="s",
                                  num_cores=_NC, num_subcores=_NS)
    sc_fn = pl.kernel(
        functools.partial(_sc_margin_body, n_rows, n_cols),
        out_type=jax.ShapeDtypeStruct((n_rows,), jnp.float32),
        mesh=mesh,
        scratch_types=[
            pltpu.VMEM((rows_per_w,), jnp.int32),
            pltpu.VMEM((rows_per_w,), jnp.int32),
            pltpu.VMEM((rows_per_w,), jnp.float32),
            pltpu.SemaphoreType.DMA,
        ],
    )
    adj = sc_fn(cosine.reshape(-1), label)

    blk_r, blk_c = 8, n_cols
    grid = (n_rows // blk_r, pl.cdiv(n_cols, blk_c))
    out = pl.pallas_call(
        functools.partial(_tc_inject_body, blk_c),
        grid=grid,
        in_specs=[
            pl.BlockSpec((blk_r, blk_c), lambda i, j: (i, j)),
            pl.BlockSpec((blk_r, 1), lambda i, j: (i, 0)),
            pl.BlockSpec((blk_r, 1), lambda i, j: (i, 0)),
        ],
        out_specs=pl.BlockSpec((blk_r, blk_c), lambda i, j: (i, j)),
        out_shape=jax.ShapeDtypeStruct((n_rows, n_cols), jnp.float32),
        compiler_params=pltpu.CompilerParams(
            dimension_semantics=("parallel", "parallel"),
        ),
    )(cosine, label.reshape(n_rows, 1), adj.reshape(n_rows, 1))
    return out
